# baseline (speedup 1.0000x reference)
import jax
import jax.numpy as jnp
from jax.experimental import pallas as pl
from jax.experimental.pallas import tpu as pltpu

BATCH = 64
IN = 2048
UNITS = 2048
NUM_HEADS = 8
WORD = 128
MEM = 256
NW = 2
NR = 4
EPS = 1e-6

GRID = 16
UB = UNITS // GRID
BB = BATCH // GRID


def _body(inp_ref, w_ref, b_ref, mem_ref, keys_ref, str_ref, ww_ref, fg_ref,
          rw_ref, plink_ref, pprec_ref, pusage_ref,
          dense_ref, cw_ref, usage_ref, link_ref, prec_ref):
    dense_ref[...] = (
        jnp.dot(inp_ref[...].astype(jnp.bfloat16),
                w_ref[...].astype(jnp.bfloat16),
                preferred_element_type=jnp.float32)
        + b_ref[...]
    )

    mem = mem_ref[...]
    keys = keys_ref[...]
    str_blk = str_ref[0]
    fg_blk = fg_ref[0]
    pu = pusage_ref[0]
    dot = jax.lax.dot_general(
        keys, mem, (((2,), (2,)), ((0,), (0,))),
        preferred_element_type=jnp.float32)
    mem_norm = jnp.sqrt(jnp.sum(mem * mem, axis=-1) + EPS)
    key_norm = jnp.sqrt(jnp.sum(keys * keys, axis=-1) + EPS)
    sim = dot / (key_norm[:, :, None] * mem_norm[:, None, :])
    x = sim * str_blk[:, :, None]
    x = x - jnp.max(x, axis=-1, keepdims=True)
    e = jnp.exp(x)
    cw_ref[...] = e / jnp.sum(e, axis=-1, keepdims=True)

    ww = ww_ref[...]
    ww_agg = 1.0 - (1.0 - ww[:, 0, :]) * (1.0 - ww[:, 1, :])
    usage_after_write = pu + (1.0 - pu) * ww_agg
    fr = 1.0 - fg_blk[:, :, None] * rw_ref[...]
    phi = fr[:, 0, :] * fr[:, 1, :] * fr[:, 2, :] * fr[:, 3, :]
    usage_ref[0] = usage_after_write * phi

    wsum = jnp.sum(ww, axis=2, keepdims=True)
    pprec = pprec_ref[...]
    prec_ref[...] = (1.0 - wsum) * pprec + ww

    wi = ww[:, :, :, None]
    wj = ww[:, :, None, :]
    pj = pprec[:, :, None, :]
    link = (1.0 - wi - wj) * plink_ref[...] + wi * pj
    row = jax.lax.broadcasted_iota(jnp.int32, (MEM, MEM), 0)
    col = jax.lax.broadcasted_iota(jnp.int32, (MEM, MEM), 1)
    mask = (row != col).astype(jnp.float32)
    link_ref[...] = link * mask[None, None, :, :]


def kernel(inputs, memory, keys, strengths, write_weights, free_gate,
           read_weights, prev_link, prev_precedence, prev_usage, W, b):
    b2 = b.reshape(1, UNITS)
    str3 = strengths.reshape(GRID, BB, NUM_HEADS)
    fg3 = free_gate.reshape(GRID, BB, NR)
    pu3 = prev_usage.reshape(GRID, BB, MEM)
    out_shapes = (
        jax.ShapeDtypeStruct((BATCH, UNITS), jnp.float32),
        jax.ShapeDtypeStruct((BATCH, NUM_HEADS, MEM), jnp.float32),
        jax.ShapeDtypeStruct((GRID, BB, MEM), jnp.float32),
        jax.ShapeDtypeStruct((BATCH, NW, MEM, MEM), jnp.float32),
        jax.ShapeDtypeStruct((BATCH, NW, MEM), jnp.float32),
    )
    in_specs = [
        pl.BlockSpec((BATCH, IN), lambda i: (0, 0)),
        pl.BlockSpec((IN, UB), lambda i: (0, i)),
        pl.BlockSpec((1, UB), lambda i: (0, i)),
        pl.BlockSpec((BB, MEM, WORD), lambda i: (i, 0, 0)),
        pl.BlockSpec((BB, NUM_HEADS, WORD), lambda i: (i, 0, 0)),
        pl.BlockSpec((1, BB, NUM_HEADS), lambda i: (i, 0, 0)),
        pl.BlockSpec((BB, NW, MEM), lambda i: (i, 0, 0)),
        pl.BlockSpec((1, BB, NR), lambda i: (i, 0, 0)),
        pl.BlockSpec((BB, NR, MEM), lambda i: (i, 0, 0)),
        pl.BlockSpec((BB, NW, MEM, MEM), lambda i: (i, 0, 0, 0)),
        pl.BlockSpec((BB, NW, MEM), lambda i: (i, 0, 0)),
        pl.BlockSpec((1, BB, MEM), lambda i: (i, 0, 0)),
    ]
    out_specs = (
        pl.BlockSpec((BATCH, UB), lambda i: (0, i)),
        pl.BlockSpec((BB, NUM_HEADS, MEM), lambda i: (i, 0, 0)),
        pl.BlockSpec((1, BB, MEM), lambda i: (i, 0, 0)),
        pl.BlockSpec((BB, NW, MEM, MEM), lambda i: (i, 0, 0, 0)),
        pl.BlockSpec((BB, NW, MEM), lambda i: (i, 0, 0)),
    )
    dense_out, cw, usage3, link, precedence = pl.pallas_call(
        _body,
        grid=(GRID,),
        in_specs=in_specs,
        out_specs=out_specs,
        out_shape=out_shapes,
        compiler_params=pltpu.CompilerParams(
            dimension_semantics=("arbitrary",),
        ),
    )(inputs, W, b2, memory, keys, str3, write_weights, fg3,
      read_weights, prev_link, prev_precedence, pu3)
    return (dense_out, cw, usage3.reshape(BATCH, MEM), link, precedence)


# speedup vs baseline: 1.1274x; 1.1274x over previous
import jax
import jax.numpy as jnp
from jax.experimental import pallas as pl
from jax.experimental.pallas import tpu as pltpu

BATCH = 64
IN = 2048
UNITS = 2048
NUM_HEADS = 8
WORD = 128
MEM = 256
NW = 2
NR = 4
EPS = 1e-6

GRID = 8
UB = UNITS // GRID
BB = BATCH // GRID


def _body(inp_ref, w_ref, b_ref, mem_ref, keys_ref, str_ref, ww_ref, fg_ref,
          rw_ref, plink_ref, pprec_ref, pusage_ref,
          dense_ref, cw_ref, usage_ref, link_ref, prec_ref):
    dense_ref[...] = (
        jnp.dot(inp_ref[...].astype(jnp.bfloat16),
                w_ref[...].astype(jnp.bfloat16),
                preferred_element_type=jnp.float32)
        + b_ref[...]
    )

    mem = mem_ref[...]
    keys = keys_ref[...]
    dot = jax.lax.dot_general(
        keys, mem, (((2,), (2,)), ((0,), (0,))),
        preferred_element_type=jnp.float32)
    mem_norm = jnp.sqrt(jnp.sum(mem * mem, axis=-1) + EPS)
    key_norm = jnp.sqrt(jnp.sum(keys * keys, axis=-1) + EPS)
    sim = dot / (key_norm[:, :, None] * mem_norm[:, None, :])
    x = sim * str_ref[...][:, :, None]
    x = x - jnp.max(x, axis=-1, keepdims=True)
    e = jnp.exp(x)
    cw_ref[...] = e / jnp.sum(e, axis=-1, keepdims=True)

    ww = ww_ref[...]
    ww_agg = 1.0 - (1.0 - ww[:, 0, :]) * (1.0 - ww[:, 1, :])
    pu = pusage_ref[...]
    usage_after_write = pu + (1.0 - pu) * ww_agg
    fr = 1.0 - fg_ref[...][:, :, None] * rw_ref[...]
    phi = fr[:, 0, :] * fr[:, 1, :] * fr[:, 2, :] * fr[:, 3, :]
    usage_ref[...] = usage_after_write * phi

    wsum = jnp.sum(ww, axis=2, keepdims=True)
    pprec = pprec_ref[...]
    prec_ref[...] = (1.0 - wsum) * pprec + ww

    wi = ww[:, :, :, None]
    wj = ww[:, :, None, :]
    pj = pprec[:, :, None, :]
    link = (1.0 - wi - wj) * plink_ref[...] + wi * pj
    row = jax.lax.broadcasted_iota(jnp.int32, (MEM, MEM), 0)
    col = jax.lax.broadcasted_iota(jnp.int32, (MEM, MEM), 1)
    mask = (row != col).astype(jnp.float32)
    link_ref[...] = link * mask[None, None, :, :]


def kernel(inputs, memory, keys, strengths, write_weights, free_gate,
           read_weights, prev_link, prev_precedence, prev_usage, W, b):
    b2 = b.reshape(1, UNITS)
    out_shapes = (
        jax.ShapeDtypeStruct((BATCH, UNITS), jnp.float32),
        jax.ShapeDtypeStruct((BATCH, NUM_HEADS, MEM), jnp.float32),
        jax.ShapeDtypeStruct((BATCH, MEM), jnp.float32),
        jax.ShapeDtypeStruct((BATCH, NW, MEM, MEM), jnp.float32),
        jax.ShapeDtypeStruct((BATCH, NW, MEM), jnp.float32),
    )
    in_specs = [
        pl.BlockSpec((BATCH, IN), lambda i: (0, 0)),
        pl.BlockSpec((IN, UB), lambda i: (0, i)),
        pl.BlockSpec((1, UB), lambda i: (0, i)),
        pl.BlockSpec((BB, MEM, WORD), lambda i: (i, 0, 0)),
        pl.BlockSpec((BB, NUM_HEADS, WORD), lambda i: (i, 0, 0)),
        pl.BlockSpec((BB, NUM_HEADS), lambda i: (i, 0)),
        pl.BlockSpec((BB, NW, MEM), lambda i: (i, 0, 0)),
        pl.BlockSpec((BB, NR), lambda i: (i, 0)),
        pl.BlockSpec((BB, NR, MEM), lambda i: (i, 0, 0)),
        pl.BlockSpec((BB, NW, MEM, MEM), lambda i: (i, 0, 0, 0)),
        pl.BlockSpec((BB, NW, MEM), lambda i: (i, 0, 0)),
        pl.BlockSpec((BB, MEM), lambda i: (i, 0)),
    ]
    out_specs = (
        pl.BlockSpec((BATCH, UB), lambda i: (0, i)),
        pl.BlockSpec((BB, NUM_HEADS, MEM), lambda i: (i, 0, 0)),
        pl.BlockSpec((BB, MEM), lambda i: (i, 0)),
        pl.BlockSpec((BB, NW, MEM, MEM), lambda i: (i, 0, 0, 0)),
        pl.BlockSpec((BB, NW, MEM), lambda i: (i, 0, 0)),
    )
    return pl.pallas_call(
        _body,
        grid=(GRID,),
        in_specs=in_specs,
        out_specs=out_specs,
        out_shape=out_shapes,
        compiler_params=pltpu.CompilerParams(
            dimension_semantics=("arbitrary",),
        ),
    )(inputs, W, b2, memory, keys, strengths, write_weights, free_gate,
      read_weights, prev_link, prev_precedence, prev_usage)
